# Initial kernel scaffold; baseline (speedup 1.0000x reference)
import jax
import jax.numpy as jnp
from jax.experimental import pallas as pl


def kernel(z, codebook):
    raise NotImplementedError("write your pallas kernel here")


# speedup vs baseline: 1.2073x; 1.2073x over previous
import functools

import jax
import jax.numpy as jnp
from jax import lax
from jax.experimental import pallas as pl
from jax.experimental.pallas import tpu as pltpu
from jax.experimental.pallas import tpu_sc as plsc

N_CODES = 8192
DIM = 32
N_VECS = 8192
BN = 1024
CK = 2048
BETA = 0.25


def _tc_body(zsq_ref, cbsq_ref, z_ref, cb_ref, ids_ref, loss_ref):
    z = z_ref[...]
    zbf = z.astype(jnp.bfloat16)
    zsq = zsq_ref[...]
    run_min = jnp.full((BN, 1), jnp.inf, dtype=jnp.float32)
    run_arg = jnp.zeros((BN, 1), dtype=jnp.int32)
    true_min = jnp.full((BN, 1), jnp.inf, dtype=jnp.float32)
    for c in range(N_CODES // CK):
        cbc = cb_ref[c * CK:(c + 1) * CK, :]
        cbsq = cbsq_ref[:, c * CK:(c + 1) * CK]
        mm = lax.dot_general(zbf, cbc,
                             (((1,), (1,)), ((), ())),
                             preferred_element_type=jnp.float32)
        dist = (zsq - 2.0 * mm) + cbsq
        mval = jnp.min(dist, axis=1, keepdims=True)
        col = lax.broadcasted_iota(jnp.int32, dist.shape, 1)
        marg = jnp.min(jnp.where(dist == mval, col, jnp.int32(2**30)),
                       axis=1, keepdims=True)
        upd = mval < run_min
        run_arg = jnp.where(upd, marg + jnp.int32(c * CK), run_arg)
        run_min = jnp.where(
            upd, mval.astype(jnp.bfloat16).astype(jnp.float32), run_min)
        true_min = jnp.minimum(true_min, mval)
    ids_ref[...] = run_arg
    part = jnp.sum(true_min, axis=(0, 1), keepdims=True)
    i = pl.program_id(0)

    @pl.when(i == 0)
    def _init():
        loss_ref[...] = part

    @pl.when(i != 0)
    def _acc():
        loss_ref[...] += part


def _tc_argmin(zsq, cbsq, z_f, cb):
    return pl.pallas_call(
        _tc_body,
        grid=(N_VECS // BN,),
        in_specs=[
            pl.BlockSpec((BN, 1), lambda i: (i, 0)),
            pl.BlockSpec((1, N_CODES), lambda i: (0, 0)),
            pl.BlockSpec((BN, DIM), lambda i: (i, 0)),
            pl.BlockSpec((N_CODES, DIM), lambda i: (0, 0)),
        ],
        out_specs=[
            pl.BlockSpec((BN, 1), lambda i: (i, 0)),
            pl.BlockSpec((1, 1), lambda i: (0, 0)),
        ],
        out_shape=[
            jax.ShapeDtypeStruct((N_VECS, 1), jnp.int32),
            jax.ShapeDtypeStruct((1, 1), jnp.float32),
        ],
    )(zsq, cbsq, z_f, cb)


_NC = 2
_NS = 16
_NW = _NC * _NS
_BPW = N_VECS // _NW
_GCHUNK = 128
_DPAD = 128


@functools.lru_cache(maxsize=None)
def _make_sc_gather():
    @functools.partial(
        pl.kernel,
        mesh=plsc.VectorSubcoreMesh(core_axis_name="c", subcore_axis_name="s"),
        out_type=jax.ShapeDtypeStruct((N_VECS, _DPAD), jnp.float32),
        scratch_types=[
            pltpu.VMEM((_GCHUNK,), jnp.int32),
            pltpu.VMEM((_GCHUNK, _DPAD), jnp.float32),
            pltpu.SemaphoreType.DMA,
        ],
    )
    def _sc_gather(cb_hbm, ids_hbm, out_hbm, idx_v, rows_v, sem):
        wid = lax.axis_index("s") * _NC + lax.axis_index("c")
        base = wid * _BPW
        for j in range(_BPW // _GCHUNK):
            off = base + j * _GCHUNK
            pltpu.sync_copy(ids_hbm.at[pl.ds(off, _GCHUNK)], idx_v)
            pltpu.async_copy(cb_hbm.at[idx_v], rows_v, sem).wait()
            pltpu.sync_copy(rows_v, out_hbm.at[pl.ds(off, _GCHUNK)])

    return _sc_gather


def kernel(z, codebook):
    z_f = z.reshape(-1, DIM)
    zsq = jnp.sum(z ** 2, axis=-1).reshape(-1, 1)
    cbsq = jnp.sum(codebook ** 2, axis=-1)[None, :]
    ids2, lacc = _tc_argmin(zsq, cbsq, z_f, codebook)
    ids = ids2.reshape(-1)
    cb_pad = jnp.pad(codebook, ((0, 0), (0, _DPAD - DIM)))
    z_q = _make_sc_gather()(cb_pad, ids)[:, :DIM].reshape(z.shape)
    loss = lacc[0, 0] * jnp.float32((1.0 + BETA) / (N_VECS * DIM))
    return (z, z_q, loss, ids)


# speedup vs baseline: 1.3175x; 1.0913x over previous
import functools

import jax
import jax.numpy as jnp
from jax import lax
from jax.experimental import pallas as pl
from jax.experimental.pallas import tpu as pltpu
from jax.experimental.pallas import tpu_sc as plsc

N_CODES = 8192
DIM = 32
N_VECS = 8192
BN = 1024
CK = 2048
BETA = 0.25


def _tc_body(zsq_ref, cbsq_ref, z_ref, cb_ref, ids_ref, loss_ref):
    z = z_ref[...]
    zbf = z.astype(jnp.bfloat16)
    zsq = zsq_ref[...]
    run_min = jnp.full((BN, 1), jnp.inf, dtype=jnp.float32)
    run_arg = jnp.zeros((BN, 1), dtype=jnp.int32)
    true_min = jnp.full((BN, 1), jnp.inf, dtype=jnp.float32)
    col = lax.broadcasted_iota(jnp.int32, (BN, CK), 1).astype(jnp.float32)
    for c in range(N_CODES // CK):
        cbc = cb_ref[c * CK:(c + 1) * CK, :]
        cbsq = cbsq_ref[:, c * CK:(c + 1) * CK]
        mm = lax.dot_general(zbf, cbc,
                             (((1,), (1,)), ((), ())),
                             preferred_element_type=jnp.float32)
        dist = (zsq + mm) + cbsq
        mval = jnp.min(dist, axis=1, keepdims=True)
        marg_f = jnp.min(jnp.where(dist == mval, col, jnp.float32(2**30)),
                         axis=1, keepdims=True)
        marg = marg_f.astype(jnp.int32)
        upd = mval < run_min
        run_arg = jnp.where(upd, marg + jnp.int32(c * CK), run_arg)
        run_min = jnp.where(
            upd, mval.astype(jnp.bfloat16).astype(jnp.float32), run_min)
        true_min = jnp.minimum(true_min, mval)
    ids_ref[...] = run_arg
    part = jnp.sum(true_min, axis=(0, 1), keepdims=True)
    i = pl.program_id(0)

    @pl.when(i == 0)
    def _init():
        loss_ref[...] = part

    @pl.when(i != 0)
    def _acc():
        loss_ref[...] += part


def _tc_argmin(zsq, cbsq, z_f, cb):
    return pl.pallas_call(
        _tc_body,
        grid=(N_VECS // BN,),
        in_specs=[
            pl.BlockSpec((BN, 1), lambda i: (i, 0)),
            pl.BlockSpec((1, N_CODES), lambda i: (0, 0)),
            pl.BlockSpec((BN, DIM), lambda i: (i, 0)),
            pl.BlockSpec((N_CODES, DIM), lambda i: (0, 0)),
        ],
        out_specs=[
            pl.BlockSpec((BN, 1), lambda i: (i, 0)),
            pl.BlockSpec((1, 1), lambda i: (0, 0)),
        ],
        out_shape=[
            jax.ShapeDtypeStruct((N_VECS, 1), jnp.int32),
            jax.ShapeDtypeStruct((1, 1), jnp.float32),
        ],
    )(zsq, cbsq, z_f, cb)


_NC = 2
_NS = 16
_NW = _NC * _NS
_BPW = N_VECS // _NW
_GCHUNK = 128
_DPAD = 128


@functools.lru_cache(maxsize=None)
def _make_sc_gather():
    @functools.partial(
        pl.kernel,
        mesh=plsc.VectorSubcoreMesh(core_axis_name="c", subcore_axis_name="s"),
        out_type=jax.ShapeDtypeStruct((N_VECS, _DPAD), jnp.float32),
        scratch_types=[
            pltpu.VMEM((_GCHUNK,), jnp.int32),
            pltpu.VMEM((_GCHUNK, _DPAD), jnp.float32),
            pltpu.SemaphoreType.DMA,
        ],
    )
    def _sc_gather(cb_hbm, ids_hbm, out_hbm, idx_v, rows_v, sem):
        wid = lax.axis_index("s") * _NC + lax.axis_index("c")
        base = wid * _BPW
        for j in range(_BPW // _GCHUNK):
            off = base + j * _GCHUNK
            pltpu.sync_copy(ids_hbm.at[pl.ds(off, _GCHUNK)], idx_v)
            pltpu.async_copy(cb_hbm.at[idx_v], rows_v, sem).wait()
            pltpu.sync_copy(rows_v, out_hbm.at[pl.ds(off, _GCHUNK)])

    return _sc_gather


def kernel(z, codebook):
    z_f = z.reshape(-1, DIM)
    zsq = jnp.sum(z ** 2, axis=-1).reshape(-1, 1)
    cbsq = jnp.sum(codebook ** 2, axis=-1)[None, :]
    ids2, lacc = _tc_argmin(zsq, cbsq, z_f, -2.0 * codebook)
    ids = ids2.reshape(-1)
    cb_pad = jnp.pad(codebook, ((0, 0), (0, _DPAD - DIM)))
    z_q = _make_sc_gather()(cb_pad, ids)[:, :DIM].reshape(z.shape)
    loss = lacc[0, 0] * jnp.float32((1.0 + BETA) / (N_VECS * DIM))
    return (z, z_q, loss, ids)
